# Initial kernel scaffold; baseline (speedup 1.0000x reference)
import jax
import jax.numpy as jnp
from jax.experimental import pallas as pl


def kernel(eeg_semantic, eeg_embedding):
    raise NotImplementedError("write your pallas kernel here")


# speedup vs baseline: 1.2093x; 1.2093x over previous
import functools

import jax
import jax.numpy as jnp
from jax import lax
from jax.experimental import pallas as pl
from jax.experimental.pallas import tpu as pltpu
from jax.experimental.pallas import tpu_sc as plsc

ROWS_PER_BLOCK = 1024
CB_CHUNK = 1024


def _argmin_body(z_ref, e_ref, e2_ref, z2_ref, idx_ref):
    z_blk = z_ref[...].astype(jnp.bfloat16).astype(jnp.float32)
    z2 = z2_ref[...]
    n_embed = e_ref.shape[0]
    half = n_embed // 2
    half_min = []
    half_idx = []
    for h in range(2):
        run_min = None
        run_idx = None
        for c in range(half // CB_CHUNK):
            j0 = h * half + c * CB_CHUNK
            e_chunk = e_ref[pl.ds(j0, CB_CHUNK), :]
            e2_chunk = e2_ref[:, pl.ds(j0, CB_CHUNK)]
            mm = lax.dot_general(z_blk, e_chunk, (((1,), (1,)), ((), ())),
                                 preferred_element_type=jnp.float32)
            d = (e2_chunk + z2) - 2.0 * mm
            cmin = jnp.min(d, axis=1, keepdims=True)
            iota = lax.broadcasted_iota(jnp.int32, d.shape, 1) + j0
            cidx = jnp.min(jnp.where(d == cmin, iota, n_embed), axis=1)
            if run_min is None:
                run_min, run_idx = cmin[:, 0], cidx
            else:
                better = cmin[:, 0] < run_min
                run_idx = jnp.where(better, cidx, run_idx)
                run_min = jnp.minimum(cmin[:, 0], run_min)
        half_min.append(run_min)
        half_idx.append(run_idx)
    m_l_bf = half_min[0].astype(jnp.bfloat16).astype(jnp.float32)
    take_right = half_min[1] < m_l_bf
    idx_ref[0, 0, :] = jnp.where(take_right, half_idx[1], half_idx[0])


def _nearest_indices(z, emb, e2, z2):
    n_rows, d_dim = z.shape
    n_embed = emb.shape[0]
    grid = n_rows // ROWS_PER_BLOCK
    out = pl.pallas_call(
        _argmin_body,
        grid=(grid,),
        in_specs=[
            pl.BlockSpec((ROWS_PER_BLOCK, d_dim), lambda i: (i, 0)),
            pl.BlockSpec((n_embed, d_dim), lambda i: (0, 0)),
            pl.BlockSpec((1, n_embed), lambda i: (0, 0)),
            pl.BlockSpec((ROWS_PER_BLOCK, 1), lambda i: (i, 0)),
        ],
        out_specs=pl.BlockSpec((1, 1, ROWS_PER_BLOCK), lambda i: (i, 0, 0)),
        out_shape=jax.ShapeDtypeStruct((grid, 1, ROWS_PER_BLOCK), jnp.int32),
    )(z, emb, e2, z2)
    return out.reshape(-1)


def _gather_rows_sc(table, idx):
    n_rows = idx.shape[0]
    d_dim = table.shape[1]
    info = plsc.get_sparse_core_info()
    nw = info.num_cores * info.num_subcores
    b_per_w = n_rows // nw
    k = 128
    n_chunks = b_per_w // k
    idx3 = idx.reshape(nw, n_chunks, k)
    mesh = plsc.VectorSubcoreMesh(core_axis_name="c", subcore_axis_name="s")

    @functools.partial(
        pl.kernel, mesh=mesh,
        out_type=jax.ShapeDtypeStruct((n_rows, d_dim), jnp.float32),
        compiler_params=pltpu.CompilerParams(use_tc_tiling_on_sc=False),
        scratch_types=[
            pltpu.VMEM((n_chunks, k), jnp.int32),
            pltpu.VMEM((b_per_w, d_dim), jnp.float32),
            pltpu.SemaphoreType.DMA,
        ],
    )
    def gk(table_hbm, idx_hbm, out_hbm, idx_v, rows_v, sem):
        wid = lax.axis_index("s") * info.num_cores + lax.axis_index("c")
        pltpu.sync_copy(idx_hbm.at[wid], idx_v)
        copies = [
            pltpu.async_copy(table_hbm.at[idx_v.at[j]],
                             rows_v.at[pl.ds(j * k, k)], sem)
            for j in range(n_chunks)
        ]
        for cp in copies:
            cp.wait()
        pltpu.sync_copy(rows_v, out_hbm.at[pl.ds(wid * b_per_w, b_per_w)])

    return gk(table, idx3)


def kernel(eeg_semantic, eeg_embedding):
    b, t, d_dim = eeg_semantic.shape
    z = eeg_semantic.reshape(-1, d_dim)
    z2 = jnp.sum(z ** 2, axis=1, keepdims=True)
    e2 = jnp.sum(eeg_embedding ** 2, axis=1)[None, :]
    idx = _nearest_indices(z, eeg_embedding, e2, z2)
    q = _gather_rows_sc(eeg_embedding, idx).reshape(eeg_semantic.shape)
    return eeg_semantic + lax.stop_gradient(q - eeg_semantic)


# speedup vs baseline: 1.2592x; 1.0413x over previous
import functools

import jax
import jax.numpy as jnp
from jax import lax
from jax.experimental import pallas as pl
from jax.experimental.pallas import tpu as pltpu
from jax.experimental.pallas import tpu_sc as plsc

ROWS_PER_BLOCK = 1024
CB_CHUNK = 1024


def _argmin_body(z_ref, e_ref, e2_ref, z2_ref, idx_ref):
    z_blk = (2.0 * z_ref[...]).astype(jnp.bfloat16).astype(jnp.float32)
    z2 = z2_ref[...]
    n_embed = e_ref.shape[0]
    half = n_embed // 2
    half_min = []
    half_idx = []
    for h in range(2):
        run_min = None
        run_idx = None
        for c in range(half // CB_CHUNK):
            j0 = h * half + c * CB_CHUNK
            e_chunk = e_ref[pl.ds(j0, CB_CHUNK), :]
            e2_chunk = e2_ref[:, pl.ds(j0, CB_CHUNK)]
            mm2 = lax.dot_general(z_blk, e_chunk, (((1,), (1,)), ((), ())),
                                  preferred_element_type=jnp.float32)
            d = (e2_chunk + z2) - mm2
            cmin = jnp.min(d, axis=1, keepdims=True)
            iota = lax.broadcasted_iota(jnp.int32, d.shape, 1) + j0
            cidx = jnp.min(jnp.where(d == cmin, iota, n_embed), axis=1)
            if run_min is None:
                run_min, run_idx = cmin[:, 0], cidx
            else:
                better = cmin[:, 0] < run_min
                run_idx = jnp.where(better, cidx, run_idx)
                run_min = jnp.minimum(cmin[:, 0], run_min)
        half_min.append(run_min)
        half_idx.append(run_idx)
    m_l_bf = half_min[0].astype(jnp.bfloat16).astype(jnp.float32)
    take_right = half_min[1] < m_l_bf
    idx_ref[0, 0, :] = jnp.where(take_right, half_idx[1], half_idx[0])


def _nearest_indices(z, emb, e2, z2):
    n_rows, d_dim = z.shape
    n_embed = emb.shape[0]
    grid = n_rows // ROWS_PER_BLOCK
    out = pl.pallas_call(
        _argmin_body,
        grid=(grid,),
        in_specs=[
            pl.BlockSpec((ROWS_PER_BLOCK, d_dim), lambda i: (i, 0)),
            pl.BlockSpec((n_embed, d_dim), lambda i: (0, 0)),
            pl.BlockSpec((1, n_embed), lambda i: (0, 0)),
            pl.BlockSpec((ROWS_PER_BLOCK, 1), lambda i: (i, 0)),
        ],
        out_specs=pl.BlockSpec((1, 1, ROWS_PER_BLOCK), lambda i: (i, 0, 0)),
        out_shape=jax.ShapeDtypeStruct((grid, 1, ROWS_PER_BLOCK), jnp.int32),
    )(z, emb, e2, z2)
    return out.reshape(-1)


def _gather_rows_sc(table, idx):
    n_rows = idx.shape[0]
    d_dim = table.shape[1]
    info = plsc.get_sparse_core_info()
    nw = info.num_cores * info.num_subcores
    b_per_w = n_rows // nw
    k = 128
    n_chunks = b_per_w // k
    idx3 = idx.reshape(nw, n_chunks, k)
    mesh = plsc.VectorSubcoreMesh(core_axis_name="c", subcore_axis_name="s")

    @functools.partial(
        pl.kernel, mesh=mesh,
        out_type=jax.ShapeDtypeStruct((n_rows, d_dim), jnp.float32),
        compiler_params=pltpu.CompilerParams(use_tc_tiling_on_sc=False),
        scratch_types=[
            pltpu.VMEM((n_chunks, k), jnp.int32),
            pltpu.VMEM((b_per_w, d_dim), jnp.float32),
            pltpu.SemaphoreType.DMA,
        ],
    )
    def gk(table_hbm, idx_hbm, out_hbm, idx_v, rows_v, sem):
        wid = lax.axis_index("s") * info.num_cores + lax.axis_index("c")
        pltpu.sync_copy(idx_hbm.at[wid], idx_v)
        copies = [
            pltpu.async_copy(table_hbm.at[idx_v.at[j]],
                             rows_v.at[pl.ds(j * k, k)], sem)
            for j in range(n_chunks)
        ]
        for cp in copies:
            cp.wait()
        pltpu.sync_copy(rows_v, out_hbm.at[pl.ds(wid * b_per_w, b_per_w)])

    return gk(table, idx3)


def kernel(eeg_semantic, eeg_embedding):
    b, t, d_dim = eeg_semantic.shape
    z = eeg_semantic.reshape(-1, d_dim)
    z2 = jnp.sum(z ** 2, axis=1, keepdims=True)
    e2 = jnp.sum(eeg_embedding ** 2, axis=1)[None, :]
    idx = _nearest_indices(z, eeg_embedding, e2, z2)
    q = _gather_rows_sc(eeg_embedding, idx).reshape(eeg_semantic.shape)
    return eeg_semantic + lax.stop_gradient(q - eeg_semantic)


# speedup vs baseline: 1.2656x; 1.0050x over previous
import functools

import jax
import jax.numpy as jnp
from jax import lax
from jax.experimental import pallas as pl
from jax.experimental.pallas import tpu as pltpu
from jax.experimental.pallas import tpu_sc as plsc

ROWS_PER_BLOCK = 1024
CB_CHUNK = 2048


def _argmin_body(z_ref, e_ref, e2_ref, z2_ref, idx_ref):
    z_blk = (2.0 * z_ref[...]).astype(jnp.bfloat16).astype(jnp.float32)
    z2 = z2_ref[...]
    n_embed = e_ref.shape[0]
    half = n_embed // 2
    half_min = []
    half_idx = []
    for h in range(2):
        run_min = None
        run_idx = None
        for c in range(half // CB_CHUNK):
            j0 = h * half + c * CB_CHUNK
            e_chunk = e_ref[pl.ds(j0, CB_CHUNK), :]
            e2_chunk = e2_ref[:, pl.ds(j0, CB_CHUNK)]
            mm2 = lax.dot_general(z_blk, e_chunk, (((1,), (1,)), ((), ())),
                                  preferred_element_type=jnp.float32)
            d = (e2_chunk + z2) - mm2
            cmin = jnp.min(d, axis=1, keepdims=True)
            iota = lax.broadcasted_iota(jnp.int32, d.shape, 1) + j0
            cidx = jnp.min(jnp.where(d == cmin, iota, n_embed), axis=1)
            if run_min is None:
                run_min, run_idx = cmin[:, 0], cidx
            else:
                better = cmin[:, 0] < run_min
                run_idx = jnp.where(better, cidx, run_idx)
                run_min = jnp.minimum(cmin[:, 0], run_min)
        half_min.append(run_min)
        half_idx.append(run_idx)
    m_l_bf = half_min[0].astype(jnp.bfloat16).astype(jnp.float32)
    take_right = half_min[1] < m_l_bf
    idx_ref[0, 0, :] = jnp.where(take_right, half_idx[1], half_idx[0])


def _nearest_indices(z, emb, e2, z2):
    n_rows, d_dim = z.shape
    n_embed = emb.shape[0]
    grid = n_rows // ROWS_PER_BLOCK
    out = pl.pallas_call(
        _argmin_body,
        grid=(grid,),
        in_specs=[
            pl.BlockSpec((ROWS_PER_BLOCK, d_dim), lambda i: (i, 0)),
            pl.BlockSpec((n_embed, d_dim), lambda i: (0, 0)),
            pl.BlockSpec((1, n_embed), lambda i: (0, 0)),
            pl.BlockSpec((ROWS_PER_BLOCK, 1), lambda i: (i, 0)),
        ],
        out_specs=pl.BlockSpec((1, 1, ROWS_PER_BLOCK), lambda i: (i, 0, 0)),
        out_shape=jax.ShapeDtypeStruct((grid, 1, ROWS_PER_BLOCK), jnp.int32),
    )(z, emb, e2, z2)
    return out.reshape(-1)


def _gather_rows_sc(table, idx):
    n_rows = idx.shape[0]
    d_dim = table.shape[1]
    info = plsc.get_sparse_core_info()
    nw = info.num_cores * info.num_subcores
    b_per_w = n_rows // nw
    k = 128
    n_chunks = b_per_w // k
    idx3 = idx.reshape(nw, n_chunks, k)
    mesh = plsc.VectorSubcoreMesh(core_axis_name="c", subcore_axis_name="s")

    @functools.partial(
        pl.kernel, mesh=mesh,
        out_type=jax.ShapeDtypeStruct((n_rows, d_dim), jnp.float32),
        compiler_params=pltpu.CompilerParams(use_tc_tiling_on_sc=False),
        scratch_types=[
            pltpu.VMEM((n_chunks, k), jnp.int32),
            pltpu.VMEM((b_per_w, d_dim), jnp.float32),
            pltpu.SemaphoreType.DMA,
        ],
    )
    def gk(table_hbm, idx_hbm, out_hbm, idx_v, rows_v, sem):
        wid = lax.axis_index("s") * info.num_cores + lax.axis_index("c")
        pltpu.sync_copy(idx_hbm.at[wid], idx_v)
        copies = [
            pltpu.async_copy(table_hbm.at[idx_v.at[j]],
                             rows_v.at[pl.ds(j * k, k)], sem)
            for j in range(n_chunks)
        ]
        for cp in copies:
            cp.wait()
        pltpu.sync_copy(rows_v, out_hbm.at[pl.ds(wid * b_per_w, b_per_w)])

    return gk(table, idx3)


def kernel(eeg_semantic, eeg_embedding):
    b, t, d_dim = eeg_semantic.shape
    z = eeg_semantic.reshape(-1, d_dim)
    z2 = jnp.sum(z ** 2, axis=1, keepdims=True)
    e2 = jnp.sum(eeg_embedding ** 2, axis=1)[None, :]
    idx = _nearest_indices(z, eeg_embedding, e2, z2)
    q = _gather_rows_sc(eeg_embedding, idx).reshape(eeg_semantic.shape)
    return eeg_semantic + lax.stop_gradient(q - eeg_semantic)
